# baseline (speedup 1.0000x reference)
import functools

import jax
import jax.numpy as jnp
import numpy as np
from jax import lax
from jax.experimental import pallas as pl
from jax.experimental.pallas import tpu as pltpu
from jax.experimental.pallas import tpu_sc as plsc

N = 10000
D = 256
HD = D // 2
E = 160000
NTILES = 16
CH = 64
EPT = 10240
EPAD = NTILES * EPT
NCHUNK = EPT // CH
NB = 4
RPT = N // NTILES
ZR = 25
SHIFT = 14

_P = np.concatenate([
    np.stack([np.arange(16), np.arange(16) + 16], axis=1).ravel() + 32 * g
    for g in range(4)])
_PERM = np.concatenate([_P, _P + HD])


def _build_spmm():
    mesh = plsc.VectorSubcoreMesh(core_axis_name="c", subcore_axis_name="s")
    scratch = (
        [pltpu.VMEM((CH, HD), jnp.bfloat16) for _ in range(NB)]
        + [pltpu.VMEM((CH, HD), jnp.float32) for _ in range(NB)]
        + [pltpu.VMEM((NB, CH), jnp.int32),
           pltpu.VMEM((NB, CH), jnp.int32),
           pltpu.VMEM((NB, CH), jnp.int32),
           pltpu.VMEM((NB, CH), jnp.float32),
           pltpu.VMEM_SHARED((N, HD), jnp.float32)]
        + [pltpu.SemaphoreType.DMA for _ in range(3 * NB)]
    )

    @functools.partial(
        pl.kernel, mesh=mesh,
        out_type=jax.ShapeDtypeStruct((2, N, HD), jnp.float32),
        scratch_types=scratch,
        compiler_params=pltpu.CompilerParams(use_tc_tiling_on_sc=False,
                                             needs_layout_passes=False))
    def spmm(x_hbm, rc_hbm, vals_hbm, out_hbm,
             ga, gb, gc, gd, sa, sb, sc, sd, rcr, colr, rowr, valr, acc,
             g0, g1, g2, g3, c0, c1, c2, c3, m0, m1, m2, m3):
        gbufs = (ga, gb, gc, gd)
        sbufs = (sa, sb, sc, sd)
        gs = (g0, g1, g2, g3)
        cs = (c0, c1, c2, c3)
        ms = (m0, m1, m2, m3)
        c = lax.axis_index("c")
        s = lax.axis_index("s")

        def m_start(j, b):
            pltpu.async_copy(rc_hbm.at[s, j], rcr.at[b], ms[b])
            pltpu.async_copy(vals_hbm.at[s, j], valr.at[b], ms[b])

        def m_wait(j, b):
            pltpu.make_async_copy(rc_hbm.at[s, j], rcr.at[b], ms[b]).wait()
            pltpu.make_async_copy(vals_hbm.at[s, j], valr.at[b], ms[b]).wait()

        def unpack(b):
            for k in range(CH // 16):
                sl = pl.ds(16 * k, 16)
                rc = rcr[b, sl]
                rowr[b, sl] = lax.shift_right_logical(rc, SHIFT)
                colr[b, sl] = lax.bitwise_and(rc, (1 << SHIFT) - 1)

        def g_start(j, b):
            pltpu.async_copy(x_hbm.at[c].at[colr.at[b]], gbufs[b], gs[b])

        def g_wait(j, b):
            pltpu.make_async_copy(x_hbm.at[c].at[colr.at[b]],
                                  gbufs[b], gs[b]).wait()

        def c_start(j, b):
            pltpu.async_copy(sbufs[b], acc.at[rowr.at[b]], cs[b], add=True)

        def c_wait(j, b):
            pltpu.make_async_copy(sbufs[b], acc.at[rowr.at[b]], cs[b]).wait()

        for b in range(NB):
            m_start(b, b)
        for b in range(2):
            m_wait(b, b)
            unpack(b)
            g_start(b, b)

        zero = jnp.zeros((16,), jnp.float32)
        for r in range(ZR):
            for k in range(HD // 16):
                sa[r, pl.ds(16 * k, 16)] = zero

        def zbody(t, carry):
            pltpu.sync_copy(sa.at[pl.ds(0, ZR)],
                            acc.at[pl.ds(s * RPT + t * ZR, ZR)])
            return carry
        lax.fori_loop(0, RPT // ZR, zbody, 0)

        plsc.subcore_barrier()

        def scale(jv, b):
            def sbody(jj, carry):
                vv = valr[b, pl.ds(jj * 16, 16)]
                for j2 in range(16):
                    vj = vv.at[jnp.full((16,), j2, jnp.int32)].get(
                        mode="promise_in_bounds")
                    row = jj * 16 + j2
                    for k in range(HD // 32):
                        g = gbufs[b][row, pl.ds(32 * k, 32)]
                        lo, hi = plsc.unpack(
                            g, format=plsc.PackFormat.INTERLEAVED)
                        sbufs[b][row, pl.ds(32 * k, 16)] = lo * vj
                        sbufs[b][row, pl.ds(32 * k + 16, 16)] = hi * vj
                return carry
            lax.fori_loop(0, CH // 16, sbody, 0)

        def chunk(jv, b):
            b2 = (b + 2) % NB
            g_wait(jv, b)
            scale(jv, b)
            c_start(jv, b)

            @pl.when(jv >= 2)
            def _():
                c_wait(jv - 2, b2)

            @pl.when(jv + 2 < NCHUNK)
            def _():
                m_wait(jv + 2, b2)
                unpack(b2)
                g_start(jv + 2, b2)

            @pl.when(jv + 4 < NCHUNK)
            def _():
                m_start(jv + 4, b)

        def gbody(t, carry):
            for b in range(NB):
                chunk(t * NB + b, b)
            return carry
        lax.fori_loop(0, NCHUNK // NB, gbody, 0)
        c_wait(NCHUNK - 2, (NCHUNK - 2) % NB)
        c_wait(NCHUNK - 1, (NCHUNK - 1) % NB)

        plsc.subcore_barrier()
        pltpu.sync_copy(acc.at[pl.ds(s * RPT, RPT)],
                        out_hbm.at[c, pl.ds(s * RPT, RPT)])

    return spmm


_BR = 1000


def _dense_mid(xh, w, b):
    def body(x_ref, w_ref, b_ref, o_ref):
        x = jnp.concatenate([x_ref[0], x_ref[1]], axis=1)
        y = lax.dot_general(x, w_ref[...], (((1,), (1,)), ((), ())),
                            preferred_element_type=jnp.float32)
        o_ref[0] = jnp.maximum(y + b_ref[...], 0.0).astype(jnp.bfloat16)

    return pl.pallas_call(
        body,
        grid=(N // _BR, 2),
        in_specs=[pl.BlockSpec((2, _BR, HD), lambda i, j: (0, i, 0)),
                  pl.BlockSpec((HD, D), lambda i, j: (j, 0)),
                  pl.BlockSpec((1, HD), lambda i, j: (0, j))],
        out_specs=pl.BlockSpec((1, _BR, HD), lambda i, j: (j, i, 0)),
        out_shape=jax.ShapeDtypeStruct((2, N, HD), jnp.bfloat16),
    )(xh, w, b.reshape(1, D))


def _dense_final(xh, w, b):
    def body(x_ref, w_ref, b_ref, o_ref):
        x = jnp.concatenate([x_ref[0], x_ref[1]], axis=1)
        y = lax.dot_general(x, w_ref[...], (((1,), (1,)), ((), ())),
                            preferred_element_type=jnp.float32)
        y = y + b_ref[...]
        nrm = jnp.sqrt(jnp.sum(y * y, axis=1, keepdims=True))
        o_ref[...] = y / jnp.maximum(nrm, 1e-12)

    return pl.pallas_call(
        body,
        grid=(N // _BR,),
        in_specs=[pl.BlockSpec((2, _BR, HD), lambda i: (0, i, 0)),
                  pl.BlockSpec((D, D), lambda i: (0, 0)),
                  pl.BlockSpec((1, D), lambda i: (0, 0))],
        out_specs=pl.BlockSpec((_BR, D), lambda i: (i, 0)),
        out_shape=jax.ShapeDtypeStruct((N, D), jnp.float32),
    )(xh, w, b.reshape(1, D))


def kernel(edge_index, edge_values, emb, W1, b1, W2, b2):
    rows = edge_index[0].astype(jnp.int32)
    cols = edge_index[1].astype(jnp.int32)
    rc = (rows << SHIFT) | cols
    rc = jnp.concatenate([rc, jnp.zeros(EPAD - E, jnp.int32)])
    rc = rc.reshape(NTILES, NCHUNK, CH)
    vals3 = jnp.concatenate([edge_values, jnp.zeros(EPAD - E, jnp.float32)])
    vals3 = vals3.reshape(NTILES, NCHUNK, CH)

    perm = jnp.asarray(_PERM)
    embp = emb[:, perm].astype(jnp.bfloat16)
    embh = jnp.stack([embp[:, :HD], embp[:, HD:]])
    w1p = W1[perm]
    b1p = b1[perm]
    spmm = _build_spmm()

    zh = spmm(embh, rc, vals3)
    zh = _dense_mid(zh, w1p, b1p)
    zh = spmm(zh, rc, vals3)
    return _dense_final(zh, W2, b2)


# speedup vs baseline: 2.2586x; 2.2586x over previous
import functools

import jax
import jax.numpy as jnp
from jax import lax
from jax.experimental import pallas as pl
from jax.experimental.pallas import tpu as pltpu
from jax.experimental.pallas import tpu_sc as plsc

N = 10000
D = 256
HD = D // 2
E = 160000
NTILES = 16
EPT = E // NTILES
CH = 80
NCHUNK = EPT // CH
NB = 4
RPT = N // NTILES
ZR = 25
SHIFT = 14


def _build_spmm():
    mesh = plsc.VectorSubcoreMesh(core_axis_name="c", subcore_axis_name="s")
    scratch = (
        [pltpu.VMEM((CH, HD), jnp.float32) for _ in range(NB)]
        + [pltpu.VMEM((NB, CH), jnp.int32),
           pltpu.VMEM((NB, CH), jnp.int32),
           pltpu.VMEM((NB, CH), jnp.int32),
           pltpu.VMEM((NB, CH), jnp.float32),
           pltpu.VMEM((ZR, HD), jnp.float32),
           pltpu.VMEM_SHARED((N, HD), jnp.float32)]
        + [pltpu.SemaphoreType.DMA for _ in range(3 * NB)]
    )

    @functools.partial(
        pl.kernel, mesh=mesh,
        out_type=jax.ShapeDtypeStruct((2, N, HD), jnp.float32),
        scratch_types=scratch,
        compiler_params=pltpu.CompilerParams(use_tc_tiling_on_sc=False))
    def spmm(x_hbm, rc_hbm, vals_hbm, out_hbm,
             b0, b1, b2, b3, rcr, colr, rowr, valr, zbuf, acc,
             g0, g1, g2, g3, c0, c1, c2, c3, m0, m1, m2, m3):
        bufs = (b0, b1, b2, b3)
        gs = (g0, g1, g2, g3)
        cs = (c0, c1, c2, c3)
        ms = (m0, m1, m2, m3)
        c = lax.axis_index("c")
        s = lax.axis_index("s")

        def m_start(j, b):
            pltpu.async_copy(rc_hbm.at[s, j], rcr.at[b], ms[b])
            pltpu.async_copy(vals_hbm.at[s, j], valr.at[b], ms[b])

        def m_wait(j, b):
            pltpu.make_async_copy(rc_hbm.at[s, j], rcr.at[b], ms[b]).wait()
            pltpu.make_async_copy(vals_hbm.at[s, j], valr.at[b], ms[b]).wait()

        def unpack(b):
            for k in range(CH // 16):
                sl = pl.ds(16 * k, 16)
                rc = rcr[b, sl]
                rowr[b, sl] = lax.shift_right_logical(rc, SHIFT)
                colr[b, sl] = lax.bitwise_and(rc, (1 << SHIFT) - 1)

        def g_start(j, b):
            pltpu.async_copy(x_hbm.at[c].at[colr.at[b]], bufs[b], gs[b])

        def g_wait(j, b):
            pltpu.make_async_copy(x_hbm.at[c].at[colr.at[b]],
                                  bufs[b], gs[b]).wait()

        def c_start(j, b):
            pltpu.async_copy(bufs[b], acc.at[rowr.at[b]], cs[b], add=True)

        def c_wait(j, b):
            pltpu.make_async_copy(bufs[b], acc.at[rowr.at[b]], cs[b]).wait()

        for b in range(NB):
            m_start(b, b)
        for b in range(2):
            m_wait(b, b)
            unpack(b)
            g_start(b, b)

        zero = jnp.zeros((16,), jnp.float32)
        for r in range(ZR):
            for k in range(HD // 16):
                zbuf[r, pl.ds(16 * k, 16)] = zero

        def zbody(t, carry):
            pltpu.sync_copy(zbuf, acc.at[pl.ds(s * RPT + t * ZR, ZR)])
            return carry
        lax.fori_loop(0, RPT // ZR, zbody, 0)

        plsc.subcore_barrier()

        def scale(jv, b):
            def sbody(jj, carry):
                vv = valr[b, pl.ds(jj * 16, 16)]
                for j2 in range(16):
                    vj = vv.at[jnp.full((16,), j2, jnp.int32)].get(
                        mode="promise_in_bounds")
                    row = jj * 16 + j2
                    for k in range(HD // 16):
                        sl = pl.ds(16 * k, 16)
                        bufs[b][row, sl] = bufs[b][row, sl] * vj
                return carry
            lax.fori_loop(0, CH // 16, sbody, 0)

        def chunk(jv, b):
            b2 = (b + 2) % NB
            g_wait(jv, b)
            scale(jv, b)
            c_start(jv, b)

            @pl.when(jv >= 2)
            def _():
                c_wait(jv - 2, b2)

            @pl.when(jv + 2 < NCHUNK)
            def _():
                m_wait(jv + 2, b2)
                unpack(b2)
                g_start(jv + 2, b2)

            @pl.when(jv + 4 < NCHUNK)
            def _():
                m_start(jv + 4, b)

        def gbody(t, carry):
            for b in range(NB):
                chunk(t * NB + b, b)
            return carry
        lax.fori_loop(0, NCHUNK // NB, gbody, 0)
        for j in range(NB * (NCHUNK // NB), NCHUNK):
            chunk(j, j % NB)
        c_wait(NCHUNK - 2, (NCHUNK - 2) % NB)
        c_wait(NCHUNK - 1, (NCHUNK - 1) % NB)

        plsc.subcore_barrier()
        pltpu.sync_copy(acc.at[pl.ds(s * RPT, RPT)],
                        out_hbm.at[c, pl.ds(s * RPT, RPT)])

    return spmm


_BR = 1000


def _dense_mid(xh, w, b):
    def body(x_ref, w_ref, b_ref, o_ref):
        x = jnp.concatenate([x_ref[0], x_ref[1]], axis=1)
        y = lax.dot_general(x, w_ref[...], (((1,), (1,)), ((), ())),
                            preferred_element_type=jnp.float32)
        o_ref[0] = jnp.maximum(y + b_ref[...], 0.0)

    return pl.pallas_call(
        body,
        grid=(N // _BR, 2),
        in_specs=[pl.BlockSpec((2, _BR, HD), lambda i, j: (0, i, 0)),
                  pl.BlockSpec((HD, D), lambda i, j: (j, 0)),
                  pl.BlockSpec((1, HD), lambda i, j: (0, j))],
        out_specs=pl.BlockSpec((1, _BR, HD), lambda i, j: (j, i, 0)),
        out_shape=jax.ShapeDtypeStruct((2, N, HD), jnp.float32),
    )(xh, w, b.reshape(1, D))


def _dense_final(xh, w, b):
    def body(x_ref, w_ref, b_ref, o_ref):
        x = jnp.concatenate([x_ref[0], x_ref[1]], axis=1)
        y = lax.dot_general(x, w_ref[...], (((1,), (1,)), ((), ())),
                            preferred_element_type=jnp.float32)
        y = y + b_ref[...]
        nrm = jnp.sqrt(jnp.sum(y * y, axis=1, keepdims=True))
        o_ref[...] = y / jnp.maximum(nrm, 1e-12)

    return pl.pallas_call(
        body,
        grid=(N // _BR,),
        in_specs=[pl.BlockSpec((2, _BR, HD), lambda i: (0, i, 0)),
                  pl.BlockSpec((D, D), lambda i: (0, 0)),
                  pl.BlockSpec((1, D), lambda i: (0, 0))],
        out_specs=pl.BlockSpec((_BR, D), lambda i: (i, 0)),
        out_shape=jax.ShapeDtypeStruct((N, D), jnp.float32),
    )(xh, w, b.reshape(1, D))


def kernel(edge_index, edge_values, emb, W1, b1, W2, b2):
    rows = edge_index[0].astype(jnp.int32)
    cols = edge_index[1].astype(jnp.int32)
    rc = ((rows << SHIFT) | cols).reshape(NTILES, NCHUNK, CH)
    vals3 = edge_values.reshape(NTILES, NCHUNK, CH)
    embh = jnp.stack([emb[:, :HD], emb[:, HD:]])
    spmm = _build_spmm()

    zh = spmm(embh, rc, vals3)
    zh = _dense_mid(zh, W1, b1)
    zh = spmm(zh, rc, vals3)
    return _dense_final(zh, W2, b2)
